# baseline (speedup 1.0000x reference)
import jax
import jax.numpy as jnp
from jax import lax
from jax.experimental import pallas as pl
from jax.experimental.pallas import tpu as pltpu
from jax.experimental.pallas import tpu_sc as plsc

F = 32
B = 16384
NC, NS, L = 2, 16, 16
NW = NC * NS
BPW = B // NW
CHUNK = 128
NCHUNK = BPW // CHUNK
RPS = 128 // F


def _body(sup_hbm, off_hbm, xr_hbm, yr_hbm, out_hbm,
          sup_v, off_v, xu_v, yi_v, yj_v, out_v, sem):
    wid = lax.axis_index("s") * NC + lax.axis_index("c")

    pltpu.sync_copy(sup_hbm.at[wid], sup_v)
    pltpu.sync_copy(off_hbm.at[wid], off_v)

    lane = lax.iota(jnp.int32, L)

    def chunk_body(c, carry):
        cps = [
            pltpu.async_copy(xr_hbm.at[sup_v.at[0, c]], xu_v, sem),
            pltpu.async_copy(yr_hbm.at[sup_v.at[1, c]], yi_v, sem),
            pltpu.async_copy(yr_hbm.at[sup_v.at[2, c]], yj_v, sem),
        ]
        for cp in cps:
            cp.wait()

        def group(g, carry2):
            base = g * L
            ouv = off_v[0, c, pl.ds(base, L)]
            oiv = off_v[1, c, pl.ds(base, L)]
            ojv = off_v[2, c, pl.ds(base, L)]
            acc = jnp.zeros((L,), jnp.float32)
            for k in range(L):
                b = base + k
                ou = ouv[k]
                oi = oiv[k]
                oj = ojv[k]
                p = jnp.zeros((L,), jnp.float32)
                for h in range(F // L):
                    hh = h * L
                    p = p + xu_v[b, pl.ds(ou + hh, L)] * (
                        yi_v[b, pl.ds(oi + hh, L)] - yj_v[b, pl.ds(oj + hh, L)])
                s = jnp.sum(p)
                acc = jnp.where(lane == k, s, acc)
            out_v[pl.ds(c * CHUNK + base, L)] = acc
            return carry2

        lax.fori_loop(0, CHUNK // L, group, 0)
        return carry

    lax.fori_loop(0, NCHUNK, chunk_body, 0)
    pltpu.sync_copy(out_v, out_hbm.at[pl.ds(wid * BPW, BPW)])


def kernel(user, item_i, item_j, x, y):
    mesh = plsc.VectorSubcoreMesh(core_axis_name="c", subcore_axis_name="s",
                                  num_cores=NC, num_subcores=NS)
    run = pl.kernel(
        _body,
        out_type=jax.ShapeDtypeStruct((B,), jnp.float32),
        mesh=mesh,
        compiler_params=pltpu.CompilerParams(needs_layout_passes=False,
                                             use_tc_tiling_on_sc=True),
        scratch_types=[
            pltpu.VMEM((3, NCHUNK, CHUNK), jnp.int32),
            pltpu.VMEM((3, NCHUNK, CHUNK), jnp.int32),
            pltpu.VMEM((CHUNK, 128), jnp.float32),
            pltpu.VMEM((CHUNK, 128), jnp.float32),
            pltpu.VMEM((CHUNK, 128), jnp.float32),
            pltpu.VMEM((BPW,), jnp.float32),
            pltpu.SemaphoreType.DMA,
        ],
    )
    idx = jnp.stack([user.astype(jnp.int32),
                     item_i.astype(jnp.int32),
                     item_j.astype(jnp.int32)]).reshape(3, NW, NCHUNK, CHUNK)
    sup = jnp.transpose(idx // RPS, (1, 0, 2, 3))
    off = jnp.transpose((idx % RPS) * F, (1, 0, 2, 3))
    xr = x.reshape(x.shape[0] // RPS, 128)
    yr = y.reshape(y.shape[0] // RPS, 128)
    return run(sup, off, xr, yr)


# speedup vs baseline: 2.6087x; 2.6087x over previous
import jax
import jax.numpy as jnp
from jax import lax
from jax.experimental import pallas as pl
from jax.experimental.pallas import tpu as pltpu
from jax.experimental.pallas import tpu_sc as plsc

F = 32
B = 16384
NC, NS, L = 2, 16, 16
NW = NC * NS
BPW = B // NW
CH = 8
NCH = BPW // CH


def _body(uij_hbm, xT_hbm, yT_hbm, out_hbm,
          idx_v, bufx, bufy, bufz, out_v, sem):
    wid = lax.axis_index("s") * NC + lax.axis_index("c")
    lane = lax.iota(jnp.int32, L)
    fidx0 = lax.iota(jnp.int32, L)
    fidx1 = fidx0 + L
    pltpu.sync_copy(uij_hbm.at[wid], idx_v)

    def chunk(c, acc):
        iv0 = idx_v[c, pl.ds(0, L)]
        iv1 = idx_v[c, pl.ds(CH, L)]

        def blocks(k):
            u = iv0[k]
            i_ = iv1[k]
            j_ = iv1[CH + k]
            au = pl.multiple_of((u >> 7) * 128, 128)
            ai = pl.multiple_of((i_ >> 7) * 128, 128)
            aj = pl.multiple_of((j_ >> 7) * 128, 128)
            return (
                pltpu.make_async_copy(xT_hbm.at[:, pl.ds(au, 128)], bufx.at[k], sem),
                pltpu.make_async_copy(yT_hbm.at[:, pl.ds(ai, 128)], bufy.at[k], sem),
                pltpu.make_async_copy(yT_hbm.at[:, pl.ds(aj, 128)], bufz.at[k], sem),
            )

        for k in range(CH):
            for cp in blocks(k):
                cp.start()
        for k in range(CH):
            for cp in blocks(k):
                cp.wait()

        for k in range(CH):
            lu = jnp.full((L,), iv0[k] & 127, jnp.int32)
            li = jnp.full((L,), iv1[k] & 127, jnp.int32)
            lj = jnp.full((L,), iv1[CH + k] & 127, jnp.int32)
            bvec = jnp.full((L,), k, jnp.int32)
            p = jnp.zeros((L,), jnp.float32)
            for fidx in (fidx0, fidx1):
                xu = plsc.load_gather(bufx, [bvec, fidx, lu])
                yi = plsc.load_gather(bufy, [bvec, fidx, li])
                yj = plsc.load_gather(bufz, [bvec, fidx, lj])
                p = p + xu * (yi - yj)
            s = jnp.sum(p)
            acc = jnp.where(lane == ((c % 2) * CH + k), s, acc)

        @pl.when(c % 2 == 1)
        def _():
            out_v[pl.ds((c // 2) * L, L)] = acc

        return acc

    lax.fori_loop(0, NCH, chunk, jnp.zeros((L,), jnp.float32))
    pltpu.sync_copy(out_v, out_hbm.at[pl.ds(wid * BPW, BPW)])


def kernel(user, item_i, item_j, x, y):
    mesh = plsc.VectorSubcoreMesh(core_axis_name="c", subcore_axis_name="s",
                                  num_cores=NC, num_subcores=NS)
    run = pl.kernel(
        _body,
        out_type=jax.ShapeDtypeStruct((B,), jnp.float32),
        mesh=mesh,
        compiler_params=pltpu.CompilerParams(needs_layout_passes=False,
                                             use_tc_tiling_on_sc=True),
        scratch_types=[
            pltpu.VMEM((NCH, 3 * CH), jnp.int32),
            pltpu.VMEM((CH, F, 128), jnp.float32),
            pltpu.VMEM((CH, F, 128), jnp.float32),
            pltpu.VMEM((CH, F, 128), jnp.float32),
            pltpu.VMEM((BPW,), jnp.float32),
            pltpu.SemaphoreType.DMA,
        ],
    )
    idx = jnp.stack([user.astype(jnp.int32),
                     item_i.astype(jnp.int32),
                     item_j.astype(jnp.int32)]).reshape(3, NW, NCH, CH)
    uij = jnp.transpose(idx, (1, 2, 0, 3)).reshape(NW, NCH, 3 * CH)
    return run(uij, x.T, y.T)
